# baseline (speedup 1.0000x reference)
import functools

import jax
import jax.numpy as jnp
from jax import lax
from jax.experimental import pallas as pl
from jax.experimental.pallas import tpu as pltpu
from jax.experimental.pallas import tpu_sc as plsc

VOCAB = 100000
DIM = 128
BATCH = 4096
SEQ = 200
CHUNK = 128
NC = 2
NS = 16
NW = NC * NS
BPW = BATCH // NW
TPW = BPW * SEQ
CPW = TPW // CHUNK
NLANE = 16
NVEC = DIM // NLANE
NBUF = 5
GROUP = 25
ROWS_PER_GROUP = GROUP * CHUNK // SEQ
NGROUP = CPW // GROUP
UNROLL = 8


def _make_kernel():
    mesh = plsc.VectorSubcoreMesh(core_axis_name="c", subcore_axis_name="s")

    @functools.partial(
        pl.kernel,
        out_type=jax.ShapeDtypeStruct((BATCH, DIM), jnp.float32),
        mesh=mesh,
        scratch_types=[
            pltpu.VMEM((CPW, CHUNK), jnp.int32),
            pltpu.VMEM((NBUF, CHUNK, DIM), jnp.float32),
            pltpu.VMEM((BPW, DIM), jnp.float32),
            [pltpu.SemaphoreType.DMA] * NBUF,
        ],
    )
    def enc(texts_hbm, emb_hbm, out_hbm, idx_v, rows_v, out_v, sems):
        wid = lax.axis_index("s") * NC + lax.axis_index("c")
        pltpu.sync_copy(texts_hbm.at[pl.ds(wid * CPW, CPW)], idx_v)

        def start(p, ci):
            pltpu.async_copy(emb_hbm.at[idx_v.at[ci]], rows_v.at[p], sems[p])

        def wait(p, ci):
            pltpu.make_async_copy(
                emb_hbm.at[idx_v.at[ci]], rows_v.at[p], sems[p]).wait()

        for p in range(NBUF):
            start(p, p)

        scale = jnp.float32(1.0 / SEQ)

        def acc_segment(p, off, length, accs):
            def jbody(j, accs):
                accs = list(accs)
                for u in range(UNROLL):
                    for g in range(NVEC):
                        accs[g] = accs[g] + rows_v[
                            p, off + UNROLL * j + u, pl.ds(NLANE * g, NLANE)]
                return tuple(accs)

            return lax.fori_loop(0, length // UNROLL, jbody, accs)

        zeros = tuple(jnp.zeros((NLANE,), jnp.float32) for _ in range(NVEC))

        def group_body(u, carry):
            accs = zeros
            pos = 0
            for cl in range(GROUP):
                p = cl % NBUF
                ci = GROUP * u + cl
                wait(p, ci)
                while pos < CHUNK * (cl + 1):
                    seg = min(SEQ - pos % SEQ, CHUNK * (cl + 1) - pos)
                    accs = acc_segment(p, pos - CHUNK * cl, seg, accs)
                    pos += seg
                    if pos % SEQ == 0:
                        r = pos // SEQ - 1
                        t = ROWS_PER_GROUP * u + r
                        for g in range(NVEC):
                            out_v[t, pl.ds(NLANE * g, NLANE)] = accs[g] * scale
                        accs = zeros

                @pl.when(ci + NBUF < CPW)
                def _():
                    start(p, ci + NBUF)

            return carry

        lax.fori_loop(0, NGROUP, group_body, 0)
        pltpu.sync_copy(out_v, out_hbm.at[pl.ds(wid * BPW, BPW)])

    def kern(texts, embedding):
        texts_r = texts.reshape(NW * CPW, CHUNK)
        return enc(texts_r, embedding)

    return kern


kernel = _make_kernel()
